# Initial kernel scaffold; baseline (speedup 1.0000x reference)
import jax
import jax.numpy as jnp
from jax.experimental import pallas as pl


def kernel(h, jets, mask, W_msg, b_msg, W_ih, W_hh, b_ih, b_hh):
    raise NotImplementedError("write your pallas kernel here")


# speedup vs baseline: 2.7594x; 2.7594x over previous
import functools

import jax
import jax.numpy as jnp
from jax.experimental import pallas as pl
from jax.experimental.pallas import tpu as pltpu


def _gru_block(h_ref, j_ref, wm_ref, bm_ref, wim_ref, wij_ref, whh_ref,
               bih_ref, bhh_ref, out_ref, *, hid):
    hb = h_ref[...]
    msg = jnp.tanh(
        jnp.dot(hb, wm_ref[...], preferred_element_type=jnp.float32)
        + bm_ref[...])
    gi = (jnp.dot(msg, wim_ref[...], preferred_element_type=jnp.float32)
          + jnp.dot(j_ref[...], wij_ref[...],
                    preferred_element_type=jnp.float32)
          + bih_ref[...])
    gh = (jnp.dot(hb, whh_ref[...], preferred_element_type=jnp.float32)
          + bhh_ref[...])
    i_r, i_z, i_n = gi[:, :hid], gi[:, hid:2 * hid], gi[:, 2 * hid:]
    h_r, h_z, h_n = gh[:, :hid], gh[:, hid:2 * hid], gh[:, 2 * hid:]
    r = jax.nn.sigmoid(i_r + h_r)
    z = jax.nn.sigmoid(i_z + h_z)
    n = jnp.tanh(i_n + r * h_n)
    out_ref[...] = (1.0 - z) * n + z * hb


def kernel(h, jets, mask, W_msg, b_msg, W_ih, W_hh, b_ih, b_hh):
    del mask
    B, N, HID = h.shape
    FEAT = jets.shape[-1]
    M = B * N
    h2 = h.reshape(M, HID)
    j2 = jets.reshape(M, FEAT)

    Wm = W_msg.T
    Wim = W_ih[:, :HID].T
    Wij = W_ih[:, HID:].T
    Whh = W_hh.T
    bm = b_msg.reshape(1, HID)
    bih = b_ih.reshape(1, 3 * HID)
    bhh = b_hh.reshape(1, 3 * HID)

    BM = 1024
    grid = (M // BM,)

    row_spec = lambda w: pl.BlockSpec((BM, w), lambda i: (i, 0))
    full_spec = lambda a: pl.BlockSpec(a.shape, lambda i: (0, 0))

    out = pl.pallas_call(
        functools.partial(_gru_block, hid=HID),
        grid=grid,
        in_specs=[
            row_spec(HID),
            row_spec(FEAT),
            full_spec(Wm), full_spec(bm),
            full_spec(Wim), full_spec(Wij),
            full_spec(Whh), full_spec(bih), full_spec(bhh),
        ],
        out_specs=row_spec(HID),
        out_shape=jax.ShapeDtypeStruct((M, HID), jnp.float32),
        compiler_params=pltpu.CompilerParams(
            dimension_semantics=("arbitrary",),
        ),
    )(h2, j2, Wm, bm, Wim, Wij, Whh, bih, bhh)
    return out.reshape(B, N, HID)


# speedup vs baseline: 2.8209x; 1.0223x over previous
import functools

import jax
import jax.numpy as jnp
from jax.experimental import pallas as pl
from jax.experimental.pallas import tpu as pltpu


def _gru_block(h_ref, j_ref, wm_ref, bm_ref, wim_ref, wij_ref, whh_ref,
               bih_ref, bhh_ref, out_ref, *, hid):
    hb = h_ref[...]
    hb16 = hb.astype(jnp.bfloat16)
    msg = jnp.tanh(
        jnp.dot(hb16, wm_ref[...], preferred_element_type=jnp.float32)
        + bm_ref[...])
    gi = (jnp.dot(msg.astype(jnp.bfloat16), wim_ref[...],
                  preferred_element_type=jnp.float32)
          + jnp.dot(j_ref[...].astype(jnp.bfloat16), wij_ref[...],
                    preferred_element_type=jnp.float32)
          + bih_ref[...])
    gh = (jnp.dot(hb16, whh_ref[...], preferred_element_type=jnp.float32)
          + bhh_ref[...])
    i_r, i_z, i_n = gi[:, :hid], gi[:, hid:2 * hid], gi[:, 2 * hid:]
    h_r, h_z, h_n = gh[:, :hid], gh[:, hid:2 * hid], gh[:, 2 * hid:]
    r = jax.nn.sigmoid(i_r + h_r)
    z = jax.nn.sigmoid(i_z + h_z)
    n = jnp.tanh(i_n + r * h_n)
    out_ref[...] = (1.0 - z) * n + z * hb


def kernel(h, jets, mask, W_msg, b_msg, W_ih, W_hh, b_ih, b_hh):
    del mask
    B, N, HID = h.shape
    FEAT = jets.shape[-1]
    M = B * N
    h2 = h.reshape(M, HID)
    j2 = jets.reshape(M, FEAT)

    Wm = W_msg.T.astype(jnp.bfloat16)
    Wim = W_ih[:, :HID].T.astype(jnp.bfloat16)
    Wij = W_ih[:, HID:].T.astype(jnp.bfloat16)
    Whh = W_hh.T.astype(jnp.bfloat16)
    bm = b_msg.reshape(1, HID)
    bih = b_ih.reshape(1, 3 * HID)
    bhh = b_hh.reshape(1, 3 * HID)

    BM = 1024
    grid = (M // BM,)

    row_spec = lambda w: pl.BlockSpec((BM, w), lambda i: (i, 0))
    full_spec = lambda a: pl.BlockSpec(a.shape, lambda i: (0, 0))

    out = pl.pallas_call(
        functools.partial(_gru_block, hid=HID),
        grid=grid,
        in_specs=[
            row_spec(HID),
            row_spec(FEAT),
            full_spec(Wm), full_spec(bm),
            full_spec(Wim), full_spec(Wij),
            full_spec(Whh), full_spec(bih), full_spec(bhh),
        ],
        out_specs=row_spec(HID),
        out_shape=jax.ShapeDtypeStruct((M, HID), jnp.float32),
        compiler_params=pltpu.CompilerParams(
            dimension_semantics=("arbitrary",),
        ),
    )(h2, j2, Wm, bm, Wim, Wij, Whh, bih, bhh)
    return out.reshape(B, N, HID)
